# Initial kernel scaffold; baseline (speedup 1.0000x reference)
import jax
import jax.numpy as jnp
from jax.experimental import pallas as pl


def kernel(x, pos_embeddings):
    raise NotImplementedError("write your pallas kernel here")


# speedup vs baseline: 2.5659x; 2.5659x over previous
import functools

import jax
import jax.numpy as jnp
from jax import lax
from jax.experimental import pallas as pl
from jax.experimental.pallas import tpu as pltpu
from jax.experimental.pallas import tpu_sc as plsc

_BLOCK_ROWS = 2048
_SC_NUM_CORES = 2
_SC_LANES = 16


def _row0_sc_kernel(x_ref, pe_ref, out_ref, row_v, pe_v, *, batch, seq_len, d_model):
    wid = lax.axis_index("s") * _SC_NUM_CORES + lax.axis_index("c")

    @pl.when(wid < batch)
    def _fix_batch_row():
        pltpu.sync_copy(x_ref.at[wid * seq_len], row_v)
        pltpu.sync_copy(pe_ref.at[0], pe_v)
        for i in range(d_model // _SC_LANES):
            sl = pl.ds(i * _SC_LANES, _SC_LANES)
            row_v[sl] = row_v[sl] + pe_v[sl]
        pltpu.sync_copy(row_v, out_ref.at[wid])


def _pe_add_kernel(x_ref, pe_ref, row0_ref, o_ref, *, seq_len):
    pe1 = pe_ref[1, :]
    o_ref[...] = x_ref[...] + pe1[None, :]
    i = pl.program_id(0)

    @pl.when((i * _BLOCK_ROWS) % seq_len == 0)
    def _splice_row0():
        o_ref[0, :] = row0_ref[(i * _BLOCK_ROWS) // seq_len, :]


def kernel(x, pos_embeddings):
    b, s, d = x.shape
    x2 = x.reshape(b * s, d)

    row0 = pl.kernel(
        functools.partial(_row0_sc_kernel, batch=b, seq_len=s, d_model=d),
        out_type=jax.ShapeDtypeStruct((b, d), x.dtype),
        mesh=plsc.VectorSubcoreMesh(core_axis_name="c", subcore_axis_name="s"),
        scratch_types=[
            pltpu.VMEM((d,), jnp.float32),
            pltpu.VMEM((d,), jnp.float32),
        ],
    )(x2, pos_embeddings)

    out = pl.pallas_call(
        functools.partial(_pe_add_kernel, seq_len=s),
        grid=(b * s // _BLOCK_ROWS,),
        in_specs=[
            pl.BlockSpec((_BLOCK_ROWS, d), lambda i: (i, 0)),
            pl.BlockSpec((2, d), lambda i: (0, 0)),
            pl.BlockSpec((b, d), lambda i: (0, 0)),
        ],
        out_specs=pl.BlockSpec((_BLOCK_ROWS, d), lambda i: (i, 0)),
        out_shape=jax.ShapeDtypeStruct((b * s, d), x.dtype),
    )(x2, pos_embeddings, row0)
    return out.reshape(b, s, d)
